# baseline (speedup 1.0000x reference)
import jax
import jax.numpy as jnp
from jax import lax
from jax.experimental import pallas as pl
from jax.experimental.pallas import tpu as pltpu
from jax.experimental.pallas import tpu_sc as plsc

_B, _S, _V, _L = 16, 32, 1000, 256
_NC, _NS = 2, 16
_NW = _NC * _NS
_SEGS_PER_W = (_B * _S) // _NW
_MAX_W = 64
_GRP = 8


def _sc_bow(tokens_hbm, spans_hbm, fill_hbm, out_hbm, row, tk, sp, fv, sem):
    cid = lax.axis_index("c")
    sid = lax.axis_index("s")
    wid = cid * _NS + sid
    b = wid // 2

    d_tok = pltpu.async_copy(tokens_hbm.at[pl.ds(b * _L, _L)],
                             tk.at[pl.ds(0, _L)], sem)
    d_sp = pltpu.async_copy(spans_hbm.at[b, pl.ds((wid % 2) * _SEGS_PER_W,
                                                  _SEGS_PER_W)],
                            sp, sem)
    d_fill = pltpu.async_copy(fill_hbm, fv.at[pl.ds(8, 1)], sem)

    iota = lax.iota(jnp.int32, 16)
    zeros_i = jnp.zeros((16,), jnp.int32)
    zeros_f = jnp.zeros((16,), jnp.float32)
    base16 = jnp.full((16,), 16, jnp.int32)

    def zero_all():
        def zk(k, c):
            for i in range(_SEGS_PER_W):
                row[i, pl.ds(k * 16, 16)] = zeros_f
            return c
        lax.fori_loop(0, _V // 16, zk, 0)
        for i in range(_SEGS_PER_W):
            row[i, pl.ds(_V - 16, 16)] = zeros_f

    col0 = jnp.full((16,), wid // _NW, jnp.int32)

    def seg_body(j, fill_v):
        jv = jnp.full((16,), j, jnp.int32)
        s0 = plsc.load_gather(sp, [jv, col0])
        e0 = plsc.load_gather(sp, [jv, col0 + 1])
        riv = jnp.full((16,), j, jnp.int32)
        for c in range(_MAX_W // 16):
            p = s0 + (c * 16 + iota)
            m = p < e0
            tok = plsc.load_gather(tk, [p])
            plsc.store_scatter(row, [riv, tok], fill_v, mask=m)
        return fill_v

    def _out_pair(g):
        return (row.at[pl.ds(g * _GRP, _GRP)],
                out_hbm.at[pl.ds(wid * _SEGS_PER_W + g * _GRP, _GRP)])

    def out_dma_start(g):
        src, dst = _out_pair(g)
        pltpu.async_copy(src, dst, sem)

    def out_dma_wait(g):
        src, dst = _out_pair(g)
        pltpu.make_async_copy(src, dst, sem).wait()

    zero_all()

    d_tok.wait()
    d_sp.wait()
    d_fill.wait()
    fill_v = plsc.load_gather(fv, [jnp.full((16,), 8, jnp.int32)])
    for u in range(_MAX_W // 16):
        tk[pl.ds(_L + u * 16, 16)] = zeros_i

    lax.fori_loop(0, _GRP, seg_body, fill_v)
    out_dma_start(0)
    lax.fori_loop(_GRP, 2 * _GRP, seg_body, fill_v)
    out_dma_start(1)
    out_dma_wait(0)
    out_dma_wait(1)


@jax.jit
def kernel(input_tokens, lengths, span_idxs, fill_value):
    del lengths

    mesh = plsc.VectorSubcoreMesh(core_axis_name="c", subcore_axis_name="s",
                                  num_cores=_NC, num_subcores=_NS)
    run = pl.kernel(
        _sc_bow,
        out_type=jax.ShapeDtypeStruct((_B * _S, _V), jnp.float32),
        mesh=mesh,
        compiler_params=pltpu.CompilerParams(needs_layout_passes=False,
                                             use_tc_tiling_on_sc=True),
        scratch_types=[
            pltpu.VMEM((_SEGS_PER_W, _V), jnp.float32),
            pltpu.VMEM((_L + _MAX_W,), jnp.int32),
            pltpu.VMEM((_SEGS_PER_W, 2), jnp.int32),
            pltpu.VMEM((16,), jnp.float32),
            pltpu.SemaphoreType.DMA,
        ],
    )
    bow = run(input_tokens.astype(jnp.int32), span_idxs.astype(jnp.int32),
              fill_value.astype(jnp.float32))
    return bow.reshape(_B, _S, _V)


# speedup vs baseline: 1.0169x; 1.0169x over previous
import jax
import jax.numpy as jnp
from jax import lax
from jax.experimental import pallas as pl
from jax.experimental.pallas import tpu as pltpu
from jax.experimental.pallas import tpu_sc as plsc

_B, _S, _V, _L = 16, 32, 1000, 256
_NC, _NS = 2, 16
_NW = _NC * _NS
_SEGS_PER_W = (_B * _S) // _NW
_MAX_W = 64
_GRP = 8


def _sc_bow(tokens_hbm, spans_hbm, fill_hbm, out_hbm, row, tk, sp, fv, sem):
    cid = lax.axis_index("c")
    sid = lax.axis_index("s")
    wid = cid * _NS + sid
    b = wid // 2

    d_tok = pltpu.async_copy(tokens_hbm.at[pl.ds(b * _L, _L)],
                             tk.at[pl.ds(0, _L)], sem)
    d_sp = pltpu.async_copy(spans_hbm.at[pl.ds(wid * 2 * _SEGS_PER_W,
                                               2 * _SEGS_PER_W)],
                            sp.at[pl.ds(16, 2 * _SEGS_PER_W)], sem)
    d_fill = pltpu.async_copy(fill_hbm, fv.at[pl.ds(8, 1)], sem)

    iota = lax.iota(jnp.int32, 16)
    zeros_i = jnp.zeros((16,), jnp.int32)
    zeros_f = jnp.zeros((16,), jnp.float32)
    base16 = jnp.full((16,), 16, jnp.int32)

    def zero_all():
        def zk(k, c):
            for i in range(_SEGS_PER_W):
                row[i, pl.ds(k * 16, 16)] = zeros_f
            return c
        lax.fori_loop(0, _V // 16, zk, 0)
        for i in range(_SEGS_PER_W):
            row[i, pl.ds(_V - 16, 16)] = zeros_f

    def seg_body(j, fill_v):
        jv = base16 + 2 * j
        s0 = plsc.load_gather(sp, [jv])
        e0 = plsc.load_gather(sp, [jv + 1])
        riv = jnp.full((16,), j, jnp.int32)
        for c in range(_MAX_W // 16):
            p = s0 + (c * 16 + iota)
            m = p < e0
            tok = plsc.load_gather(tk, [p])
            plsc.store_scatter(row, [riv, tok], fill_v, mask=m)
        return fill_v

    def _out_pair(g):
        return (row.at[pl.ds(g * _GRP, _GRP)],
                out_hbm.at[pl.ds(wid * _SEGS_PER_W + g * _GRP, _GRP)])

    def out_dma_start(g):
        src, dst = _out_pair(g)
        pltpu.async_copy(src, dst, sem)

    def out_dma_wait(g):
        src, dst = _out_pair(g)
        pltpu.make_async_copy(src, dst, sem).wait()

    zero_all()

    d_tok.wait()
    d_sp.wait()
    d_fill.wait()
    fill_v = plsc.load_gather(fv, [jnp.full((16,), 8, jnp.int32)])
    for u in range(_MAX_W // 16):
        tk[pl.ds(_L + u * 16, 16)] = zeros_i

    lax.fori_loop(0, _GRP, seg_body, fill_v)
    out_dma_start(0)
    lax.fori_loop(_GRP, 2 * _GRP, seg_body, fill_v)
    out_dma_start(1)
    out_dma_wait(0)
    out_dma_wait(1)


@jax.jit
def kernel(input_tokens, lengths, span_idxs, fill_value):
    del lengths
    spans_flat = span_idxs.reshape(_B * _S * 2).astype(jnp.int32)

    mesh = plsc.VectorSubcoreMesh(core_axis_name="c", subcore_axis_name="s",
                                  num_cores=_NC, num_subcores=_NS)
    run = pl.kernel(
        _sc_bow,
        out_type=jax.ShapeDtypeStruct((_B * _S, _V), jnp.float32),
        mesh=mesh,
        compiler_params=pltpu.CompilerParams(needs_layout_passes=False,
                                             use_tc_tiling_on_sc=True),
        scratch_types=[
            pltpu.VMEM((_SEGS_PER_W, _V), jnp.float32),
            pltpu.VMEM((_L + _MAX_W,), jnp.int32),
            pltpu.VMEM((16 + 2 * _SEGS_PER_W,), jnp.int32),
            pltpu.VMEM((16,), jnp.float32),
            pltpu.SemaphoreType.DMA,
        ],
    )
    bow = run(input_tokens.astype(jnp.int32), spans_flat,
              fill_value.astype(jnp.float32))
    return bow.reshape(_B, _S, _V)
